# baseline (speedup 1.0000x reference)
import functools

import jax
import jax.numpy as jnp
from jax import lax
from jax.experimental import pallas as pl
from jax.experimental.pallas import tpu as pltpu
from jax.experimental.pallas import tpu_sc as plsc

_N = 10000
_NA = 5000
_N_PAD = 10240
_NA_PAD = 5120
_EIJ = 320000
_EIA = 80000
_EAI = 80000
_Q = 128
_EPS = 1e-10

_NC = 2
_NS = 16
_NW = _NC * _NS
_CH = 128

_HN = _N_PAD // 2
_HA = _NA_PAD // 2
_DUM = 128

_HI = lax.Precision.HIGHEST


def _sc_mesh():
    return plsc.VectorSubcoreMesh(
        core_axis_name="c", subcore_axis_name="s",
        num_cores=_NC, num_subcores=_NS)


def _stats_body(mi_ref, ma_ref, c_ref, w_ref, hi_ref, ha_ref):
    pi = jax.nn.softmax(mi_ref[...], axis=1)
    s_i = jnp.mean(pi, axis=0, keepdims=True)
    pa = jax.nn.softmax(ma_ref[...], axis=1)
    s_a = jnp.mean(pa, axis=0, keepdims=True)
    hi_ref[...] = (jnp.dot(s_i, c_ref[...], preferred_element_type=jnp.float32,
                           precision=_HI)
                   + jnp.mean(jnp.dot(s_a, w_ref[...],
                                      preferred_element_type=jnp.float32,
                                      precision=_HI)))
    ha_ref[...] = jnp.dot(s_i, w_ref[...], preferred_element_type=jnp.float32,
                          precision=_HI)


def _stats(marg_i, marg_a, cmat, wmat):
    return pl.pallas_call(
        _stats_body,
        out_shape=(jax.ShapeDtypeStruct((1, _Q), jnp.float32),
                   jax.ShapeDtypeStruct((1, _Q), jnp.float32)),
    )(marg_i, marg_a, cmat, wmat)


def _matlog_body(c_ref, m_ref, o_ref):
    a = c_ref[...]
    m = m_ref[...]
    a_hi = a.astype(jnp.bfloat16)
    a_lo = (a - a_hi.astype(jnp.float32)).astype(jnp.bfloat16)
    m_hi = m.astype(jnp.bfloat16)
    m_lo = (m - m_hi.astype(jnp.float32)).astype(jnp.bfloat16)
    f32 = jnp.float32
    prod = (jnp.dot(a_hi, m_hi, preferred_element_type=f32)
            + (jnp.dot(a_hi, m_lo, preferred_element_type=f32)
               + jnp.dot(a_lo, m_hi, preferred_element_type=f32)))
    o_ref[...] = jnp.log(prod + _EPS)


def _matlog(cav, mat, blk=2000):
    e = cav.shape[0]
    return pl.pallas_call(
        _matlog_body,
        grid=(e // blk,),
        in_specs=[pl.BlockSpec((blk, _Q), lambda i: (i, 0)),
                  pl.BlockSpec((_Q, _Q), lambda i: (0, 0))],
        out_specs=pl.BlockSpec((blk, _Q), lambda i: (i, 0)),
        out_shape=jax.ShapeDtypeStruct((e, _Q), jnp.float32),
    )(cav, mat)


def _combine_i_body(p2_ref, p3_ref, f_ref, hi_ref, mi_ref):
    mi_ref[...] = (p2_ref[: _N, :] + p3_ref[: _N, :] + f_ref[...]
                   - hi_ref[...])


def _combine_i(acc2, acc3, field_i, h_i):
    return pl.pallas_call(
        _combine_i_body,
        out_shape=jax.ShapeDtypeStruct((_N, _Q), jnp.float32),
    )(acc2, acc3, field_i, h_i)


def _combine_a_body(pa_ref, ha_ref, ma_ref):
    ma_ref[...] = pa_ref[: _NA, :] - ha_ref[...]


def _combine_a(acc_a, h_a):
    return pl.pallas_call(
        _combine_a_body,
        out_shape=jax.ShapeDtypeStruct((_NA, _Q), jnp.float32),
    )(acc_a, h_a)


def _softmax_body(a_ref, o_ref):
    x = a_ref[...]
    x = x - jnp.max(x, axis=1, keepdims=True)
    ex = jnp.exp(x)
    o_ref[...] = ex / jnp.sum(ex, axis=1, keepdims=True)


def _softmax_rows(a, blk=2000):
    e = a.shape[0]
    return pl.pallas_call(
        _softmax_body,
        grid=(e // blk,),
        in_specs=[pl.BlockSpec((blk, _Q), lambda i: (i, 0))],
        out_specs=pl.BlockSpec((blk, _Q), lambda i: (i, 0)),
        out_shape=jax.ShapeDtypeStruct((e, _Q), jnp.float32),
    )(a)


def _softmax_first_half_body(a_ref, o_ref):
    _softmax_body(a_ref, o_ref)


def _softmax_second_half_body(a_ref, prev_ref, o_ref):
    del prev_ref
    _softmax_body(a_ref, o_ref)


def _softmax_halves(a_lo, a_hi, blk=2000):
    h = a_lo.shape[0]
    e = h + a_hi.shape[0]
    nb = h // blk
    part = pl.pallas_call(
        _softmax_first_half_body,
        grid=(nb,),
        in_specs=[pl.BlockSpec((blk, _Q), lambda i: (i, 0))],
        out_specs=pl.BlockSpec((blk, _Q), lambda i: (i, 0)),
        out_shape=jax.ShapeDtypeStruct((e, _Q), jnp.float32),
    )(a_lo)
    return pl.pallas_call(
        _softmax_second_half_body,
        grid=(nb,),
        in_specs=[pl.BlockSpec((blk, _Q), lambda i: (i, 0)),
                  pl.BlockSpec(memory_space=pl.ANY)],
        out_specs=pl.BlockSpec((blk, _Q), lambda i: (i + nb, 0)),
        out_shape=jax.ShapeDtypeStruct((e, _Q), jnp.float32),
        input_output_aliases={1: 0},
    )(a_hi, part)


def _segsum_sc(edge_sets, half, n_pad):
    zrows = (half + _DUM) // _NS
    orows = half // _NS

    @functools.partial(
        pl.kernel,
        out_type=jax.ShapeDtypeStruct((n_pad, _Q), jnp.float32),
        mesh=_sc_mesh(),
        scratch_types=[
            pltpu.VMEM_SHARED((half + _DUM, _Q), jnp.float32),
            pltpu.VMEM((zrows, _Q), jnp.float32),
            pltpu.VMEM((1, _CH), jnp.int32),
            pltpu.VMEM((_CH, _Q), jnp.float32),
        ],
    )
    def k(*refs):
        ins = refs[:2 * len(edge_sets)]
        out_h, acc, zbuf, idx_v, rows_v = refs[2 * len(edge_sets):]
        cid = lax.axis_index("c")
        sid = lax.axis_index("s")
        lane = lax.iota(jnp.int32, 16)
        lo = cid * half

        def zrow(r, carry):
            for k16 in range(_Q // 16):
                zbuf[r, pl.ds(k16 * 16, 16)] = jnp.zeros((16,), jnp.float32)
            return carry

        lax.fori_loop(0, zrows, zrow, 0)
        pltpu.sync_copy(zbuf, acc.at[pl.ds(sid * zrows, zrows)])
        plsc.subcore_barrier()

        for j, (temp, _) in enumerate(edge_sets):
            temp_hbm = ins[2 * j]
            idx_hbm = ins[2 * j + 1]
            nchunks = temp.shape[0] // _CH
            n_w = (nchunks - sid + _NS - 1) // _NS

            def body(i, carry, temp_hbm=temp_hbm, idx_hbm=idx_hbm):
                base = (sid + i * _NS) * _CH
                pltpu.sync_copy(idx_hbm.at[pl.ds(base, _CH)], idx_v.at[0])
                pltpu.sync_copy(temp_hbm.at[pl.ds(base, _CH)], rows_v)
                for k16 in range(_CH // 16):
                    v = idx_v[0, pl.ds(k16 * 16, 16)] - lo
                    inb = (v >= 0) & (v < half)
                    dummy = half + k16 * 16 + lane
                    idx_v[0, pl.ds(k16 * 16, 16)] = jnp.where(inb, v, dummy)
                pltpu.sync_copy(rows_v, acc.at[idx_v.at[0]], add=True)
                return carry

            lax.fori_loop(0, n_w, body, 0)

        plsc.subcore_barrier()
        pltpu.sync_copy(acc.at[pl.ds(sid * orows, orows)],
                        out_h.at[pl.ds(cid * half + sid * orows, orows)])

    flat = []
    for temp, idx in edge_sets:
        flat += [temp, idx]
    return k(*flat)


def _gather_sub_sc(tblm, idxm, tblt, idxt):
    e = idxm.shape[0]
    nchunks = e // _CH

    @functools.partial(
        pl.kernel,
        out_type=jax.ShapeDtypeStruct((e, _Q), jnp.float32),
        mesh=_sc_mesh(),
        scratch_types=[
            pltpu.VMEM((2, _CH), jnp.int32),
            pltpu.VMEM((2, _CH), jnp.int32),
            pltpu.VMEM((_CH, _Q), jnp.float32),
            pltpu.VMEM((_CH, _Q), jnp.float32),
            pltpu.VMEM((_CH, _Q), jnp.float32),
            pltpu.VMEM((_CH, _Q), jnp.float32),
            pltpu.SemaphoreType.DMA,
            pltpu.SemaphoreType.DMA,
            pltpu.SemaphoreType.DMA,
            pltpu.SemaphoreType.DMA,
            pltpu.SemaphoreType.DMA,
            pltpu.SemaphoreType.DMA,
        ],
    )
    def k(tm_hbm, im_hbm, tt_hbm, it_hbm, out_h,
          idx_m, idx_t, rows_m0, rows_m1, rows_t0, rows_t1,
          gm0, gm1, gt0, gt1, w0, w1):
        cid = lax.axis_index("c")
        sid = lax.axis_index("s")
        wid = sid * _NC + cid
        rows_m = (rows_m0, rows_m1)
        rows_t = (rows_t0, rows_t1)
        gm = (gm0, gm1)
        gt = (gt0, gt1)
        ws = (w0, w1)

        n_w = (nchunks - wid + _NW - 1) // _NW
        n_pair = (n_w + 1) // 2

        def pair(p, carry):
            for b in range(2):
                i_dyn = 2 * p + b

                @pl.when(i_dyn < n_w)
                def _():
                    base = (wid + i_dyn * _NW) * _CH
                    pltpu.sync_copy(im_hbm.at[pl.ds(base, _CH)], idx_m.at[b])
                    pltpu.sync_copy(it_hbm.at[pl.ds(base, _CH)], idx_t.at[b])

                    @pl.when(p > 0)
                    def _():
                        pltpu.make_async_copy(
                            rows_m[b], out_h.at[pl.ds(base, _CH)],
                            ws[b]).wait()

                    pltpu.async_copy(tm_hbm.at[idx_m.at[b]], rows_m[b], gm[b])
                    pltpu.async_copy(tt_hbm.at[idx_t.at[b]], rows_t[b], gt[b])

            for b in range(2):
                i_dyn = 2 * p + b

                @pl.when(i_dyn < n_w)
                def _():
                    base = (wid + i_dyn * _NW) * _CH
                    pltpu.make_async_copy(tm_hbm.at[idx_m.at[b]],
                                          rows_m[b], gm[b]).wait()
                    pltpu.make_async_copy(tt_hbm.at[idx_t.at[b]],
                                          rows_t[b], gt[b]).wait()

                    def sub_row(r, c2):
                        for k16 in range(_Q // 16):
                            sl = pl.ds(k16 * 16, 16)
                            rows_m[b][r, sl] = (rows_m[b][r, sl]
                                                - rows_t[b][r, sl])
                        return c2

                    lax.fori_loop(0, _CH, sub_row, 0)
                    pltpu.async_copy(rows_m[b],
                                     out_h.at[pl.ds(base, _CH)], ws[b])

            return carry

        lax.fori_loop(0, n_pair, pair, 0)
        for b in range(2):
            pltpu.make_async_copy(rows_m[b], out_h.at[pl.ds(0, _CH)],
                                  ws[b]).wait()

    return k(tblm, idxm, tblt, idxt)


def kernel(marg_i, marg_a, cav_ij, cav_ia, cav_ai, C, W, field_i,
           src_ij, dst_ij, node_ia, fac_ia, node_ai, fac_ai,
           indice_ij, indice_ia, indice_ai):
    i32 = jnp.int32
    src_ij = src_ij.astype(i32)
    dst_ij = dst_ij.astype(i32)
    node_ia = node_ia.astype(i32)
    fac_ia = fac_ia.astype(i32)
    node_ai = node_ai.astype(i32)
    fac_ai = fac_ai.astype(i32)
    indice_ij = indice_ij.astype(i32)
    indice_ia = indice_ia.astype(i32)
    indice_ai = indice_ai.astype(i32)

    temp1 = _matlog(cav_ia, W)
    h_i, h_a = _stats(marg_i, marg_a, C, W)
    acc_a = _segsum_sc([(temp1, fac_ia)], _HA, _NA_PAD)
    marg_a_new = _combine_a(acc_a, h_a)
    d_ai = _gather_sub_sc(marg_a_new, fac_ai, temp1, indice_ai)
    cav_ai_new = _softmax_rows(d_ai)

    cav_ij_g, _ = lax.optimization_barrier((cav_ij, temp1))
    temp2 = _matlog(cav_ij_g, C)
    temp3 = _matlog(cav_ai, W)

    acc2 = _segsum_sc([(temp2, dst_ij)], _HN, _N_PAD)
    acc3 = _segsum_sc([(temp3, node_ai)], _HN, _N_PAD)
    marg_i_new = _combine_i(acc2, acc3, field_i, h_i)

    half = _EIJ // 2
    d_ij_a = _gather_sub_sc(marg_i_new, src_ij[:half], temp2,
                            indice_ij[:half])
    d_ij_b = _gather_sub_sc(marg_i_new, src_ij[half:], temp2,
                            indice_ij[half:])
    node_ia_g, _ = lax.optimization_barrier((node_ia, d_ij_b))
    d_ia = _gather_sub_sc(marg_i_new, node_ia_g, temp3, indice_ia)
    cav_ij_new = _softmax_halves(d_ij_a, d_ij_b)
    cav_ia_new = _softmax_rows(d_ia)

    return (marg_i_new, marg_a_new, cav_ij_new, cav_ia_new, cav_ai_new)


# speedup vs baseline: 1.0088x; 1.0088x over previous
import functools

import jax
import jax.numpy as jnp
from jax import lax
from jax.experimental import pallas as pl
from jax.experimental.pallas import tpu as pltpu
from jax.experimental.pallas import tpu_sc as plsc

_N = 10000
_NA = 5000
_N_PAD = 10240
_NA_PAD = 5120
_EIJ = 320000
_EIA = 80000
_EAI = 80000
_Q = 128
_EPS = 1e-10

_NC = 2
_NS = 16
_NW = _NC * _NS
_CH = 128

_HN = _N_PAD // 2
_HA = _NA_PAD // 2
_DUM = 128

_HI = lax.Precision.HIGHEST


def _sc_mesh():
    return plsc.VectorSubcoreMesh(
        core_axis_name="c", subcore_axis_name="s",
        num_cores=_NC, num_subcores=_NS)


def _stats_body(mi_ref, ma_ref, c_ref, w_ref, hi_ref, ha_ref):
    pi = jax.nn.softmax(mi_ref[...], axis=1)
    s_i = jnp.mean(pi, axis=0, keepdims=True)
    pa = jax.nn.softmax(ma_ref[...], axis=1)
    s_a = jnp.mean(pa, axis=0, keepdims=True)
    hi_ref[...] = (jnp.dot(s_i, c_ref[...], preferred_element_type=jnp.float32,
                           precision=_HI)
                   + jnp.mean(jnp.dot(s_a, w_ref[...],
                                      preferred_element_type=jnp.float32,
                                      precision=_HI)))
    ha_ref[...] = jnp.dot(s_i, w_ref[...], preferred_element_type=jnp.float32,
                          precision=_HI)


def _stats(marg_i, marg_a, cmat, wmat):
    return pl.pallas_call(
        _stats_body,
        out_shape=(jax.ShapeDtypeStruct((1, _Q), jnp.float32),
                   jax.ShapeDtypeStruct((1, _Q), jnp.float32)),
    )(marg_i, marg_a, cmat, wmat)


def _matlog_body(c_ref, m_ref, o_ref):
    a = c_ref[...]
    m = m_ref[...]
    a_hi = a.astype(jnp.bfloat16)
    a_lo = (a - a_hi.astype(jnp.float32)).astype(jnp.bfloat16)
    m_hi = m.astype(jnp.bfloat16)
    m_lo = (m - m_hi.astype(jnp.float32)).astype(jnp.bfloat16)
    f32 = jnp.float32
    prod = (jnp.dot(a_hi, m_hi, preferred_element_type=f32)
            + (jnp.dot(a_hi, m_lo, preferred_element_type=f32)
               + jnp.dot(a_lo, m_hi, preferred_element_type=f32)))
    o_ref[...] = jnp.log(prod + _EPS)


def _matlog(cav, mat, blk=2000):
    e = cav.shape[0]
    return pl.pallas_call(
        _matlog_body,
        grid=(e // blk,),
        in_specs=[pl.BlockSpec((blk, _Q), lambda i: (i, 0)),
                  pl.BlockSpec((_Q, _Q), lambda i: (0, 0))],
        out_specs=pl.BlockSpec((blk, _Q), lambda i: (i, 0)),
        out_shape=jax.ShapeDtypeStruct((e, _Q), jnp.float32),
    )(cav, mat)


def _combine_i_body(pi_ref, f_ref, hi_ref, mi_ref):
    mi_ref[...] = pi_ref[: _N, :] + f_ref[...] - hi_ref[...]


def _combine_i(acc_i, field_i, h_i):
    return pl.pallas_call(
        _combine_i_body,
        out_shape=jax.ShapeDtypeStruct((_N, _Q), jnp.float32),
    )(acc_i, field_i, h_i)


def _combine_a_body(pa_ref, ha_ref, ma_ref):
    ma_ref[...] = pa_ref[: _NA, :] - ha_ref[...]


def _combine_a(acc_a, h_a):
    return pl.pallas_call(
        _combine_a_body,
        out_shape=jax.ShapeDtypeStruct((_NA, _Q), jnp.float32),
    )(acc_a, h_a)


def _softmax_body(a_ref, o_ref):
    x = a_ref[...]
    x = x - jnp.max(x, axis=1, keepdims=True)
    ex = jnp.exp(x)
    o_ref[...] = ex / jnp.sum(ex, axis=1, keepdims=True)


def _softmax_rows(a, blk=2000):
    e = a.shape[0]
    return pl.pallas_call(
        _softmax_body,
        grid=(e // blk,),
        in_specs=[pl.BlockSpec((blk, _Q), lambda i: (i, 0))],
        out_specs=pl.BlockSpec((blk, _Q), lambda i: (i, 0)),
        out_shape=jax.ShapeDtypeStruct((e, _Q), jnp.float32),
    )(a)


def _softmax_first_half_body(a_ref, o_ref):
    _softmax_body(a_ref, o_ref)


def _softmax_second_half_body(a_ref, prev_ref, o_ref):
    del prev_ref
    _softmax_body(a_ref, o_ref)


def _softmax_halves(a_lo, a_hi, blk=2000):
    h = a_lo.shape[0]
    e = h + a_hi.shape[0]
    nb = h // blk
    part = pl.pallas_call(
        _softmax_first_half_body,
        grid=(nb,),
        in_specs=[pl.BlockSpec((blk, _Q), lambda i: (i, 0))],
        out_specs=pl.BlockSpec((blk, _Q), lambda i: (i, 0)),
        out_shape=jax.ShapeDtypeStruct((e, _Q), jnp.float32),
    )(a_lo)
    return pl.pallas_call(
        _softmax_second_half_body,
        grid=(nb,),
        in_specs=[pl.BlockSpec((blk, _Q), lambda i: (i, 0)),
                  pl.BlockSpec(memory_space=pl.ANY)],
        out_specs=pl.BlockSpec((blk, _Q), lambda i: (i + nb, 0)),
        out_shape=jax.ShapeDtypeStruct((e, _Q), jnp.float32),
        input_output_aliases={1: 0},
    )(a_hi, part)


def _segsum_sc(edge_sets, half, n_pad):
    zrows = (half + _DUM) // _NS
    orows = half // _NS

    @functools.partial(
        pl.kernel,
        out_type=jax.ShapeDtypeStruct((n_pad, _Q), jnp.float32),
        mesh=_sc_mesh(),
        scratch_types=[
            pltpu.VMEM_SHARED((half + _DUM, _Q), jnp.float32),
            pltpu.VMEM((zrows, _Q), jnp.float32),
            pltpu.VMEM((1, _CH), jnp.int32),
            pltpu.VMEM((_CH, _Q), jnp.float32),
        ],
    )
    def k(*refs):
        ins = refs[:2 * len(edge_sets)]
        out_h, acc, zbuf, idx_v, rows_v = refs[2 * len(edge_sets):]
        cid = lax.axis_index("c")
        sid = lax.axis_index("s")
        lane = lax.iota(jnp.int32, 16)
        lo = cid * half

        def zrow(r, carry):
            for k16 in range(_Q // 16):
                zbuf[r, pl.ds(k16 * 16, 16)] = jnp.zeros((16,), jnp.float32)
            return carry

        lax.fori_loop(0, zrows, zrow, 0)
        pltpu.sync_copy(zbuf, acc.at[pl.ds(sid * zrows, zrows)])
        plsc.subcore_barrier()

        for j, (temp, _) in enumerate(edge_sets):
            temp_hbm = ins[2 * j]
            idx_hbm = ins[2 * j + 1]
            nchunks = temp.shape[0] // _CH
            n_w = (nchunks - sid + _NS - 1) // _NS

            def body(i, carry, temp_hbm=temp_hbm, idx_hbm=idx_hbm):
                base = (sid + i * _NS) * _CH
                pltpu.sync_copy(idx_hbm.at[pl.ds(base, _CH)], idx_v.at[0])
                pltpu.sync_copy(temp_hbm.at[pl.ds(base, _CH)], rows_v)
                for k16 in range(_CH // 16):
                    v = idx_v[0, pl.ds(k16 * 16, 16)] - lo
                    inb = (v >= 0) & (v < half)
                    dummy = half + k16 * 16 + lane
                    idx_v[0, pl.ds(k16 * 16, 16)] = jnp.where(inb, v, dummy)
                pltpu.sync_copy(rows_v, acc.at[idx_v.at[0]], add=True)
                return carry

            lax.fori_loop(0, n_w, body, 0)

        plsc.subcore_barrier()
        pltpu.sync_copy(acc.at[pl.ds(sid * orows, orows)],
                        out_h.at[pl.ds(cid * half + sid * orows, orows)])

    flat = []
    for temp, idx in edge_sets:
        flat += [temp, idx]
    return k(*flat)


def _gather_sub_sc(tblm, idxm, tblt, idxt):
    e = idxm.shape[0]
    nchunks = e // _CH

    @functools.partial(
        pl.kernel,
        out_type=jax.ShapeDtypeStruct((e, _Q), jnp.float32),
        mesh=_sc_mesh(),
        scratch_types=[
            pltpu.VMEM((2, _CH), jnp.int32),
            pltpu.VMEM((2, _CH), jnp.int32),
            pltpu.VMEM((_CH, _Q), jnp.float32),
            pltpu.VMEM((_CH, _Q), jnp.float32),
            pltpu.VMEM((_CH, _Q), jnp.float32),
            pltpu.VMEM((_CH, _Q), jnp.float32),
            pltpu.SemaphoreType.DMA,
            pltpu.SemaphoreType.DMA,
            pltpu.SemaphoreType.DMA,
            pltpu.SemaphoreType.DMA,
            pltpu.SemaphoreType.DMA,
            pltpu.SemaphoreType.DMA,
        ],
    )
    def k(tm_hbm, im_hbm, tt_hbm, it_hbm, out_h,
          idx_m, idx_t, rows_m0, rows_m1, rows_t0, rows_t1,
          gm0, gm1, gt0, gt1, w0, w1):
        cid = lax.axis_index("c")
        sid = lax.axis_index("s")
        wid = sid * _NC + cid
        rows_m = (rows_m0, rows_m1)
        rows_t = (rows_t0, rows_t1)
        gm = (gm0, gm1)
        gt = (gt0, gt1)
        ws = (w0, w1)

        n_w = (nchunks - wid + _NW - 1) // _NW
        n_pair = (n_w + 1) // 2

        def pair(p, carry):
            for b in range(2):
                i_dyn = 2 * p + b

                @pl.when(i_dyn < n_w)
                def _():
                    base = (wid + i_dyn * _NW) * _CH
                    pltpu.sync_copy(im_hbm.at[pl.ds(base, _CH)], idx_m.at[b])
                    pltpu.sync_copy(it_hbm.at[pl.ds(base, _CH)], idx_t.at[b])

                    @pl.when(p > 0)
                    def _():
                        pltpu.make_async_copy(
                            rows_m[b], out_h.at[pl.ds(base, _CH)],
                            ws[b]).wait()

                    pltpu.async_copy(tm_hbm.at[idx_m.at[b]], rows_m[b], gm[b])
                    pltpu.async_copy(tt_hbm.at[idx_t.at[b]], rows_t[b], gt[b])

            for b in range(2):
                i_dyn = 2 * p + b

                @pl.when(i_dyn < n_w)
                def _():
                    base = (wid + i_dyn * _NW) * _CH
                    pltpu.make_async_copy(tm_hbm.at[idx_m.at[b]],
                                          rows_m[b], gm[b]).wait()
                    pltpu.make_async_copy(tt_hbm.at[idx_t.at[b]],
                                          rows_t[b], gt[b]).wait()

                    def sub_row(r, c2):
                        for k16 in range(_Q // 16):
                            sl = pl.ds(k16 * 16, 16)
                            rows_m[b][r, sl] = (rows_m[b][r, sl]
                                                - rows_t[b][r, sl])
                        return c2

                    lax.fori_loop(0, _CH, sub_row, 0)
                    pltpu.async_copy(rows_m[b],
                                     out_h.at[pl.ds(base, _CH)], ws[b])

            return carry

        lax.fori_loop(0, n_pair, pair, 0)
        for b in range(2):
            pltpu.make_async_copy(rows_m[b], out_h.at[pl.ds(0, _CH)],
                                  ws[b]).wait()

    return k(tblm, idxm, tblt, idxt)


def kernel(marg_i, marg_a, cav_ij, cav_ia, cav_ai, C, W, field_i,
           src_ij, dst_ij, node_ia, fac_ia, node_ai, fac_ai,
           indice_ij, indice_ia, indice_ai):
    i32 = jnp.int32
    src_ij = src_ij.astype(i32)
    dst_ij = dst_ij.astype(i32)
    node_ia = node_ia.astype(i32)
    fac_ia = fac_ia.astype(i32)
    node_ai = node_ai.astype(i32)
    fac_ai = fac_ai.astype(i32)
    indice_ij = indice_ij.astype(i32)
    indice_ia = indice_ia.astype(i32)
    indice_ai = indice_ai.astype(i32)

    temp1 = _matlog(cav_ia, W)
    h_i, h_a = _stats(marg_i, marg_a, C, W)
    acc_a = _segsum_sc([(temp1, fac_ia)], _HA, _NA_PAD)
    marg_a_new = _combine_a(acc_a, h_a)
    d_ai = _gather_sub_sc(marg_a_new, fac_ai, temp1, indice_ai)
    cav_ai_new = _softmax_rows(d_ai)

    temp2 = _matlog(cav_ij, C)
    temp3 = _matlog(cav_ai, W)

    acc_i = _segsum_sc([(temp2, dst_ij), (temp3, node_ai)], _HN, _N_PAD)
    marg_i_new = _combine_i(acc_i, field_i, h_i)

    half = _EIJ // 2
    d_ij_a = _gather_sub_sc(marg_i_new, src_ij[:half], temp2,
                            indice_ij[:half])
    d_ij_b = _gather_sub_sc(marg_i_new, src_ij[half:], temp2,
                            indice_ij[half:])
    node_ia_g, _ = lax.optimization_barrier((node_ia, d_ij_b))
    d_ia = _gather_sub_sc(marg_i_new, node_ia_g, temp3, indice_ia)
    cav_ij_new = _softmax_halves(d_ij_a, d_ij_b)
    cav_ia_new = _softmax_rows(d_ia)

    return (marg_i_new, marg_a_new, cav_ij_new, cav_ia_new, cav_ai_new)


# speedup vs baseline: 1.0459x; 1.0368x over previous
import functools

import jax
import jax.numpy as jnp
from jax import lax
from jax.experimental import pallas as pl
from jax.experimental.pallas import tpu as pltpu
from jax.experimental.pallas import tpu_sc as plsc

_N = 10000
_NA = 5000
_N_PAD = 10240
_NA_PAD = 5120
_EIJ = 320000
_EIA = 80000
_EAI = 80000
_Q = 128
_EPS = 1e-10

_NC = 2
_NS = 16
_NW = _NC * _NS
_CH = 128

_HN = _N_PAD // 2
_HA = _NA_PAD // 2
_DUM = 128

_HI = lax.Precision.HIGHEST


def _sc_mesh():
    return plsc.VectorSubcoreMesh(
        core_axis_name="c", subcore_axis_name="s",
        num_cores=_NC, num_subcores=_NS)


def _stats_body(mi_ref, ma_ref, c_ref, w_ref, hi_ref, ha_ref):
    pi = jax.nn.softmax(mi_ref[...], axis=1)
    s_i = jnp.mean(pi, axis=0, keepdims=True)
    pa = jax.nn.softmax(ma_ref[...], axis=1)
    s_a = jnp.mean(pa, axis=0, keepdims=True)
    hi_ref[...] = (jnp.dot(s_i, c_ref[...], preferred_element_type=jnp.float32,
                           precision=_HI)
                   + jnp.mean(jnp.dot(s_a, w_ref[...],
                                      preferred_element_type=jnp.float32,
                                      precision=_HI)))
    ha_ref[...] = jnp.dot(s_i, w_ref[...], preferred_element_type=jnp.float32,
                          precision=_HI)


def _stats(marg_i, marg_a, cmat, wmat):
    return pl.pallas_call(
        _stats_body,
        out_shape=(jax.ShapeDtypeStruct((1, _Q), jnp.float32),
                   jax.ShapeDtypeStruct((1, _Q), jnp.float32)),
    )(marg_i, marg_a, cmat, wmat)


def _matlog_body(c_ref, m_ref, o_ref):
    a = c_ref[...]
    m = m_ref[...]
    a_hi = a.astype(jnp.bfloat16)
    a_lo = (a - a_hi.astype(jnp.float32)).astype(jnp.bfloat16)
    m_hi = m.astype(jnp.bfloat16)
    m_lo = (m - m_hi.astype(jnp.float32)).astype(jnp.bfloat16)
    f32 = jnp.float32
    prod = (jnp.dot(a_hi, m_hi, preferred_element_type=f32)
            + (jnp.dot(a_hi, m_lo, preferred_element_type=f32)
               + jnp.dot(a_lo, m_hi, preferred_element_type=f32)))
    o_ref[...] = jnp.log(prod + _EPS)


def _matlog(cav, mat, blk=2000):
    e = cav.shape[0]
    return pl.pallas_call(
        _matlog_body,
        grid=(e // blk,),
        in_specs=[pl.BlockSpec((blk, _Q), lambda i: (i, 0)),
                  pl.BlockSpec((_Q, _Q), lambda i: (0, 0))],
        out_specs=pl.BlockSpec((blk, _Q), lambda i: (i, 0)),
        out_shape=jax.ShapeDtypeStruct((e, _Q), jnp.float32),
    )(cav, mat)


def _combine_i_body(pi_ref, f_ref, hi_ref, mi_ref):
    mi_ref[...] = pi_ref[: _N, :] + f_ref[...] - hi_ref[...]


def _combine_i(acc_i, field_i, h_i):
    return pl.pallas_call(
        _combine_i_body,
        out_shape=jax.ShapeDtypeStruct((_N, _Q), jnp.float32),
    )(acc_i, field_i, h_i)


def _combine_a_body(pa_ref, ha_ref, ma_ref):
    ma_ref[...] = pa_ref[: _NA, :] - ha_ref[...]


def _combine_a(acc_a, h_a):
    return pl.pallas_call(
        _combine_a_body,
        out_shape=jax.ShapeDtypeStruct((_NA, _Q), jnp.float32),
    )(acc_a, h_a)


def _softmax_body(a_ref, o_ref):
    x = a_ref[...]
    x = x - jnp.max(x, axis=1, keepdims=True)
    ex = jnp.exp(x)
    o_ref[...] = ex / jnp.sum(ex, axis=1, keepdims=True)


def _softmax_rows(a, blk=2000):
    e = a.shape[0]
    return pl.pallas_call(
        _softmax_body,
        grid=(e // blk,),
        in_specs=[pl.BlockSpec((blk, _Q), lambda i: (i, 0))],
        out_specs=pl.BlockSpec((blk, _Q), lambda i: (i, 0)),
        out_shape=jax.ShapeDtypeStruct((e, _Q), jnp.float32),
    )(a)


def _softmax_first_half_body(a_ref, o_ref):
    _softmax_body(a_ref, o_ref)


def _softmax_second_half_body(a_ref, prev_ref, o_ref):
    del prev_ref
    _softmax_body(a_ref, o_ref)


def _softmax_halves(a_lo, a_hi, blk=2000):
    h = a_lo.shape[0]
    e = h + a_hi.shape[0]
    nb = h // blk
    part = pl.pallas_call(
        _softmax_first_half_body,
        grid=(nb,),
        in_specs=[pl.BlockSpec((blk, _Q), lambda i: (i, 0))],
        out_specs=pl.BlockSpec((blk, _Q), lambda i: (i, 0)),
        out_shape=jax.ShapeDtypeStruct((e, _Q), jnp.float32),
    )(a_lo)
    return pl.pallas_call(
        _softmax_second_half_body,
        grid=(nb,),
        in_specs=[pl.BlockSpec((blk, _Q), lambda i: (i, 0)),
                  pl.BlockSpec(memory_space=pl.ANY)],
        out_specs=pl.BlockSpec((blk, _Q), lambda i: (i + nb, 0)),
        out_shape=jax.ShapeDtypeStruct((e, _Q), jnp.float32),
        input_output_aliases={1: 0},
    )(a_hi, part)


def _segsum_sc(edge_sets, half, n_pad):
    zrows = (half + _DUM) // _NS
    orows = half // _NS

    @functools.partial(
        pl.kernel,
        out_type=jax.ShapeDtypeStruct((n_pad, _Q), jnp.float32),
        mesh=_sc_mesh(),
        scratch_types=[
            pltpu.VMEM_SHARED((half + _DUM, _Q), jnp.float32),
            pltpu.VMEM((zrows, _Q), jnp.float32),
            pltpu.VMEM((1, _CH), jnp.int32),
            pltpu.VMEM((_CH, _Q), jnp.float32),
        ],
    )
    def k(*refs):
        ins = refs[:2 * len(edge_sets)]
        out_h, acc, zbuf, idx_v, rows_v = refs[2 * len(edge_sets):]
        cid = lax.axis_index("c")
        sid = lax.axis_index("s")
        lane = lax.iota(jnp.int32, 16)
        lo = cid * half

        def zrow(r, carry):
            for k16 in range(_Q // 16):
                zbuf[r, pl.ds(k16 * 16, 16)] = jnp.zeros((16,), jnp.float32)
            return carry

        lax.fori_loop(0, zrows, zrow, 0)
        pltpu.sync_copy(zbuf, acc.at[pl.ds(sid * zrows, zrows)])
        plsc.subcore_barrier()

        for j, (temp, _) in enumerate(edge_sets):
            temp_hbm = ins[2 * j]
            idx_hbm = ins[2 * j + 1]
            nchunks = temp.shape[0] // _CH
            n_w = (nchunks - sid + _NS - 1) // _NS

            def body(i, carry, temp_hbm=temp_hbm, idx_hbm=idx_hbm):
                base = (sid + i * _NS) * _CH
                pltpu.sync_copy(idx_hbm.at[pl.ds(base, _CH)], idx_v.at[0])
                pltpu.sync_copy(temp_hbm.at[pl.ds(base, _CH)], rows_v)
                for k16 in range(_CH // 16):
                    v = idx_v[0, pl.ds(k16 * 16, 16)] - lo
                    inb = (v >= 0) & (v < half)
                    dummy = half + k16 * 16 + lane
                    idx_v[0, pl.ds(k16 * 16, 16)] = jnp.where(inb, v, dummy)
                pltpu.sync_copy(rows_v, acc.at[idx_v.at[0]], add=True)
                return carry

            lax.fori_loop(0, n_w, body, 0)

        plsc.subcore_barrier()
        pltpu.sync_copy(acc.at[pl.ds(sid * orows, orows)],
                        out_h.at[pl.ds(cid * half + sid * orows, orows)])

    flat = []
    for temp, idx in edge_sets:
        flat += [temp, idx]
    return k(*flat)


def _gather_sub_sc(tblm, idxm, tblt, idxt):
    e = idxm.shape[0]
    nchunks = e // _CH

    @functools.partial(
        pl.kernel,
        out_type=jax.ShapeDtypeStruct((e, _Q), jnp.float32),
        mesh=_sc_mesh(),
        scratch_types=[
            pltpu.VMEM((2, _CH), jnp.int32),
            pltpu.VMEM((2, _CH), jnp.int32),
            pltpu.VMEM((_CH, _Q), jnp.float32),
            pltpu.VMEM((_CH, _Q), jnp.float32),
            pltpu.VMEM((_CH, _Q), jnp.float32),
            pltpu.VMEM((_CH, _Q), jnp.float32),
            pltpu.SemaphoreType.DMA,
            pltpu.SemaphoreType.DMA,
            pltpu.SemaphoreType.DMA,
            pltpu.SemaphoreType.DMA,
            pltpu.SemaphoreType.DMA,
            pltpu.SemaphoreType.DMA,
        ],
    )
    def k(tm_hbm, im_hbm, tt_hbm, it_hbm, out_h,
          idx_m, idx_t, rows_m0, rows_m1, rows_t0, rows_t1,
          gm0, gm1, gt0, gt1, w0, w1):
        cid = lax.axis_index("c")
        sid = lax.axis_index("s")
        wid = sid * _NC + cid
        rows_m = (rows_m0, rows_m1)
        rows_t = (rows_t0, rows_t1)
        gm = (gm0, gm1)
        gt = (gt0, gt1)
        ws = (w0, w1)

        n_w = (nchunks - wid + _NW - 1) // _NW
        n_pair = (n_w + 1) // 2

        def pair(p, carry):
            for b in range(2):
                i_dyn = 2 * p + b

                @pl.when(i_dyn < n_w)
                def _():
                    base = (wid + i_dyn * _NW) * _CH
                    pltpu.sync_copy(im_hbm.at[pl.ds(base, _CH)], idx_m.at[b])
                    pltpu.sync_copy(it_hbm.at[pl.ds(base, _CH)], idx_t.at[b])

                    @pl.when(p > 0)
                    def _():
                        pltpu.make_async_copy(
                            rows_m[b], out_h.at[pl.ds(base, _CH)],
                            ws[b]).wait()

                    pltpu.async_copy(tm_hbm.at[idx_m.at[b]], rows_m[b], gm[b])
                    pltpu.async_copy(tt_hbm.at[idx_t.at[b]], rows_t[b], gt[b])

            for b in range(2):
                i_dyn = 2 * p + b

                @pl.when(i_dyn < n_w)
                def _():
                    base = (wid + i_dyn * _NW) * _CH
                    pltpu.make_async_copy(tm_hbm.at[idx_m.at[b]],
                                          rows_m[b], gm[b]).wait()
                    pltpu.make_async_copy(tt_hbm.at[idx_t.at[b]],
                                          rows_t[b], gt[b]).wait()

                    def sub_row(r, c2):
                        for k16 in range(_Q // 16):
                            sl = pl.ds(k16 * 16, 16)
                            rows_m[b][r, sl] = (rows_m[b][r, sl]
                                                - rows_t[b][r, sl])
                        return c2

                    lax.fori_loop(0, _CH, sub_row, 0)
                    pltpu.async_copy(rows_m[b],
                                     out_h.at[pl.ds(base, _CH)], ws[b])

            return carry

        lax.fori_loop(0, n_pair, pair, 0)
        for b in range(2):
            pltpu.make_async_copy(rows_m[b], out_h.at[pl.ds(0, _CH)],
                                  ws[b]).wait()

    return k(tblm, idxm, tblt, idxt)


def kernel(marg_i, marg_a, cav_ij, cav_ia, cav_ai, C, W, field_i,
           src_ij, dst_ij, node_ia, fac_ia, node_ai, fac_ai,
           indice_ij, indice_ia, indice_ai):
    i32 = jnp.int32
    src_ij = src_ij.astype(i32)
    dst_ij = dst_ij.astype(i32)
    node_ia = node_ia.astype(i32)
    fac_ia = fac_ia.astype(i32)
    node_ai = node_ai.astype(i32)
    fac_ai = fac_ai.astype(i32)
    indice_ij = indice_ij.astype(i32)
    indice_ia = indice_ia.astype(i32)
    indice_ai = indice_ai.astype(i32)

    temp1 = _matlog(cav_ia, W)
    h_i, h_a = _stats(marg_i, marg_a, C, W)
    acc_a = _segsum_sc([(temp1, fac_ia)], _HA, _NA_PAD)
    marg_a_new = _combine_a(acc_a, h_a)
    d_ai = _gather_sub_sc(marg_a_new, fac_ai, temp1, indice_ai)
    cav_ai_new = _softmax_rows(d_ai)

    temp2 = _matlog(cav_ij, C)
    temp3 = _matlog(cav_ai, W)

    acc_i = _segsum_sc([(temp2, dst_ij), (temp3, node_ai)], _HN, _N_PAD)
    marg_i_new = _combine_i(acc_i, field_i, h_i)

    half = _EIJ // 2
    d_ij_a = _gather_sub_sc(marg_i_new, src_ij[:half], temp2,
                            indice_ij[:half])
    d_ij_b = _gather_sub_sc(marg_i_new, src_ij[half:], temp2,
                            indice_ij[half:])
    d_ia = _gather_sub_sc(marg_i_new, node_ia, temp3, indice_ia)
    cav_ij_new = _softmax_halves(d_ij_a, d_ij_b)
    cav_ia_new = _softmax_rows(d_ia)

    return (marg_i_new, marg_a_new, cav_ij_new, cav_ia_new, cav_ai_new)
